# initial kernel scaffold (unmeasured)
import jax
import jax.numpy as jnp
from jax import lax
from jax.experimental import pallas as pl
from jax.experimental.pallas import tpu as pltpu

N_DEV = 8
SQ = 2048
SKV_SHARD = 2048
HQ = 8
DH = 128
DM = HQ * DH
SCALE = 0.08838834764831843
NEG = -1e9


def kernel(x, Wq, K_ext, V_ext, Wo):
    def body(x_ref, wq_ref, k_ref, v_ref, wo_ref, out_ref,
             acc_ref, ml_ref, comm_acc, comm_ml,
             send_a, recv_a, send_m, recv_m, credit_sem):
        my = lax.axis_index("i")
        left = lax.rem(my - 1 + N_DEV, N_DEV)
        right = lax.rem(my + 1, N_DEV)

        barrier = pltpu.get_barrier_semaphore()
        pl.semaphore_signal(barrier, inc=1, device_id=(left,),
                            device_id_type=pl.DeviceIdType.MESH)
        pl.semaphore_signal(barrier, inc=1, device_id=(right,),
                            device_id_type=pl.DeviceIdType.MESH)
        pl.semaphore_wait(barrier, 2)

        xb = x_ref[0].astype(jnp.bfloat16)
        wq = wq_ref[...].astype(jnp.bfloat16)
        q = lax.dot_general(xb, wq, (((1,), (0,)), ((), ())),
                            preferred_element_type=jnp.float32)

        qi = lax.broadcasted_iota(jnp.int32, (SQ, SKV_SHARD), 0)
        ki = lax.broadcasted_iota(jnp.int32, (SQ, SKV_SHARD), 1) + my * SKV_SHARD
        mask = (jnp.abs(qi - ki) <= 128) | (ki < 32) | (qi < 32)

        accs, ms, ls = [], [], []
        for h in range(HQ):
            qh = q[:, h * DH:(h + 1) * DH].astype(jnp.bfloat16)
            kh = k_ref[0, :, h, :].astype(jnp.bfloat16)
            vh = v_ref[0, :, h, :].astype(jnp.bfloat16)
            s = lax.dot_general(qh, kh, (((1,), (1,)), ((), ())),
                                preferred_element_type=jnp.float32)
            s = jnp.where(mask, s * SCALE, NEG)
            m = jnp.max(s, axis=1, keepdims=True)
            w = jnp.exp(s - m)
            l = jnp.sum(w, axis=1, keepdims=True)
            a = lax.dot_general(w.astype(jnp.bfloat16), vh,
                                (((1,), (0,)), ((), ())),
                                preferred_element_type=jnp.float32)
            accs.append(a)
            ms.append(m)
            ls.append(l)

        acc = jnp.concatenate(accs, axis=1)
        ml = jnp.concatenate(ms + ls, axis=1)
        acc_ref[...] = acc
        ml_ref[...] = ml
        comm_acc[0] = acc.astype(jnp.bfloat16)
        comm_ml[0] = ml

        for hop in range(N_DEV - 1):
            s_slot = hop % 2
            r_slot = (hop + 1) % 2
            if hop >= 2:
                pl.semaphore_wait(credit_sem, 1)
            rdma_a = pltpu.make_async_remote_copy(
                src_ref=comm_acc.at[s_slot], dst_ref=comm_acc.at[r_slot],
                send_sem=send_a.at[s_slot], recv_sem=recv_a.at[r_slot],
                device_id=(right,), device_id_type=pl.DeviceIdType.MESH)
            rdma_m = pltpu.make_async_remote_copy(
                src_ref=comm_ml.at[s_slot], dst_ref=comm_ml.at[r_slot],
                send_sem=send_m.at[s_slot], recv_sem=recv_m.at[r_slot],
                device_id=(right,), device_id_type=pl.DeviceIdType.MESH)
            rdma_a.start()
            rdma_m.start()
            rdma_a.wait()
            rdma_m.wait()

            acc_in = comm_acc[r_slot].astype(jnp.float32)
            ml_in = comm_ml[r_slot]
            acc_run = acc_ref[...]
            ml_run = ml_ref[...]
            new_acc, new_m, new_l = [], [], []
            for h in range(HQ):
                m_r = ml_run[:, h:h + 1]
                l_r = ml_run[:, HQ + h:HQ + h + 1]
                m_i = ml_in[:, h:h + 1]
                l_i = ml_in[:, HQ + h:HQ + h + 1]
                mx = jnp.maximum(m_r, m_i)
                sr = jnp.exp(m_r - mx)
                si = jnp.exp(m_i - mx)
                new_m.append(mx)
                new_l.append(l_r * sr + l_i * si)
                hs = slice(h * DH, (h + 1) * DH)
                new_acc.append(acc_run[:, hs] * sr + acc_in[:, hs] * si)
            acc_ref[...] = jnp.concatenate(new_acc, axis=1)
            ml_ref[...] = jnp.concatenate(new_m + new_l, axis=1)
            if hop < N_DEV - 3:
                pl.semaphore_signal(credit_sem, inc=1, device_id=(left,),
                                    device_id_type=pl.DeviceIdType.MESH)

        acc_run = acc_ref[...]
        ml_run = ml_ref[...]
        ctx = []
        for h in range(HQ):
            l = ml_run[:, HQ + h:HQ + h + 1]
            ctx.append((acc_run[:, h * DH:(h + 1) * DH] / l).astype(jnp.bfloat16))
        ctx = jnp.concatenate(ctx, axis=1)
        wo = wo_ref[...].astype(jnp.bfloat16)
        out = lax.dot_general(ctx, wo, (((1,), (0,)), ((), ())),
                              preferred_element_type=jnp.float32)
        out_ref[0] = out

    return pl.pallas_call(
        body,
        out_shape=jax.ShapeDtypeStruct((1, SQ, DM), jnp.float32),
        in_specs=[pl.BlockSpec(memory_space=pltpu.VMEM)] * 5,
        out_specs=pl.BlockSpec(memory_space=pltpu.VMEM),
        scratch_shapes=[
            pltpu.VMEM((SQ, DM), jnp.float32),
            pltpu.VMEM((SQ, 2 * HQ), jnp.float32),
            pltpu.VMEM((2, SQ, DM), jnp.bfloat16),
            pltpu.VMEM((2, SQ, 2 * HQ), jnp.float32),
            pltpu.SemaphoreType.DMA((2,)),
            pltpu.SemaphoreType.DMA((2,)),
            pltpu.SemaphoreType.DMA((2,)),
            pltpu.SemaphoreType.DMA((2,)),
            pltpu.SemaphoreType.REGULAR,
        ],
        compiler_params=pltpu.CompilerParams(collective_id=0),
    )(x, Wq, K_ext, V_ext, Wo)


# baseline (device time: 592401 ns/iter reference)
import jax
import jax.numpy as jnp
from jax import lax
from jax.experimental import pallas as pl
from jax.experimental.pallas import tpu as pltpu

N_DEV = 8
SQ = 2048
SKV_SHARD = 2048
HQ = 8
DH = 128
DM = HQ * DH
QB = 128
NQB = SQ // QB
SCALE = 0.08838834764831843
NEG = -1e9


def _expand_heads(s, rows):
    return jnp.broadcast_to(s[:, :, None], (rows, HQ, DH)).reshape(rows, DM)


def kernel(x, Wq, K_ext, V_ext, Wo):
    q = jnp.dot(x[0].astype(jnp.bfloat16), Wq.astype(jnp.bfloat16),
                preferred_element_type=jnp.float32)
    q_bf = q.astype(jnp.bfloat16)
    k_bf = K_ext[0].reshape(SKV_SHARD, DM).astype(jnp.bfloat16)
    v_bf = V_ext[0].reshape(SKV_SHARD, DM).astype(jnp.bfloat16)
    wo_bf = Wo.astype(jnp.bfloat16)

    def body(q_ref, k_ref, v_ref, wo_ref, out_ref,
             acc_ref, ml_ref, comm_acc, comm_ml,
             send_a, recv_a, send_m, recv_m, credit_sem):
        my = lax.axis_index("i")
        left = lax.rem(my - 1 + N_DEV, N_DEV)
        right = lax.rem(my + 1, N_DEV)

        barrier = pltpu.get_barrier_semaphore()
        pl.semaphore_signal(barrier, inc=1, device_id=(left,),
                            device_id_type=pl.DeviceIdType.MESH)
        pl.semaphore_signal(barrier, inc=1, device_id=(right,),
                            device_id_type=pl.DeviceIdType.MESH)
        pl.semaphore_wait(barrier, 2)

        def local_block(qb, _):
            rows = pl.ds(qb * QB, QB)
            qi = lax.broadcasted_iota(jnp.int32, (QB, SKV_SHARD), 0) + qb * QB
            ki = (lax.broadcasted_iota(jnp.int32, (QB, SKV_SHARD), 1)
                  + my * SKV_SHARD)
            mask = (jnp.abs(qi - ki) <= 128) | (ki < 32) | (qi < 32)
            for h in range(HQ):
                hs = slice(h * DH, (h + 1) * DH)
                s = lax.dot_general(q_ref[rows, hs], k_ref[:, hs],
                                    (((1,), (1,)), ((), ())),
                                    preferred_element_type=jnp.float32)
                s = jnp.where(mask, s * SCALE, NEG)
                m = jnp.max(s, axis=1, keepdims=True)
                w = jnp.exp(s - m)
                l = jnp.sum(w, axis=1, keepdims=True)
                a = lax.dot_general(w.astype(jnp.bfloat16), v_ref[:, hs],
                                    (((1,), (0,)), ((), ())),
                                    preferred_element_type=jnp.float32)
                acc_ref[rows, hs] = a
                ml_ref[rows, h:h + 1] = m
                ml_ref[rows, HQ + h:HQ + h + 1] = l
                comm_acc[0, rows, hs] = a.astype(jnp.bfloat16)
            return 0

        lax.fori_loop(0, NQB, local_block, 0)
        comm_ml[0] = ml_ref[...]

        for hop in range(N_DEV - 1):
            s_slot = hop % 2
            r_slot = (hop + 1) % 2
            if hop >= 2:
                pl.semaphore_wait(credit_sem, 1)
            rdma_a = pltpu.make_async_remote_copy(
                src_ref=comm_acc.at[s_slot], dst_ref=comm_acc.at[r_slot],
                send_sem=send_a.at[s_slot], recv_sem=recv_a.at[r_slot],
                device_id=(right,), device_id_type=pl.DeviceIdType.MESH)
            rdma_m = pltpu.make_async_remote_copy(
                src_ref=comm_ml.at[s_slot], dst_ref=comm_ml.at[r_slot],
                send_sem=send_m.at[s_slot], recv_sem=recv_m.at[r_slot],
                device_id=(right,), device_id_type=pl.DeviceIdType.MESH)
            rdma_a.start()
            rdma_m.start()
            rdma_a.wait()
            rdma_m.wait()

            def merge_block(qb, _):
                rows = pl.ds(qb * QB, QB)
                m_r = ml_ref[rows, 0:HQ]
                l_r = ml_ref[rows, HQ:2 * HQ]
                m_i = comm_ml[r_slot, rows, 0:HQ]
                l_i = comm_ml[r_slot, rows, HQ:2 * HQ]
                mx = jnp.maximum(m_r, m_i)
                sr = jnp.exp(m_r - mx)
                si = jnp.exp(m_i - mx)
                ml_ref[rows, 0:HQ] = mx
                ml_ref[rows, HQ:2 * HQ] = l_r * sr + l_i * si
                acc_in = comm_acc[r_slot, rows, :].astype(jnp.float32)
                acc_ref[rows, :] = (acc_ref[rows, :] * _expand_heads(sr, QB)
                                    + acc_in * _expand_heads(si, QB))
                return 0

            lax.fori_loop(0, NQB, merge_block, 0)
            if hop < N_DEV - 3:
                pl.semaphore_signal(credit_sem, inc=1, device_id=(left,),
                                    device_id_type=pl.DeviceIdType.MESH)

        def out_block(qb, _):
            rows = pl.ds(qb * QB, QB)
            linv = 1.0 / ml_ref[rows, HQ:2 * HQ]
            ctx = (acc_ref[rows, :] * _expand_heads(linv, QB)
                   ).astype(jnp.bfloat16)
            out_ref[rows, :] = lax.dot_general(
                ctx, wo_ref[...], (((1,), (0,)), ((), ())),
                preferred_element_type=jnp.float32)
            return 0

        lax.fori_loop(0, NQB, out_block, 0)

    out = pl.pallas_call(
        body,
        out_shape=jax.ShapeDtypeStruct((SQ, DM), jnp.float32),
        in_specs=[pl.BlockSpec(memory_space=pltpu.VMEM)] * 4,
        out_specs=pl.BlockSpec(memory_space=pltpu.VMEM),
        scratch_shapes=[
            pltpu.VMEM((SQ, DM), jnp.float32),
            pltpu.VMEM((SQ, 2 * HQ), jnp.float32),
            pltpu.VMEM((2, SQ, DM), jnp.bfloat16),
            pltpu.VMEM((2, SQ, 2 * HQ), jnp.float32),
            pltpu.SemaphoreType.DMA((2,)),
            pltpu.SemaphoreType.DMA((2,)),
            pltpu.SemaphoreType.DMA((2,)),
            pltpu.SemaphoreType.DMA((2,)),
            pltpu.SemaphoreType.REGULAR,
        ],
        compiler_params=pltpu.CompilerParams(
            collective_id=0, vmem_limit_bytes=60 * 1024 * 1024),
    )(q_bf, k_bf, v_bf, wo_bf)
    return out.reshape(1, SQ, DM)


# device time: 138917 ns/iter; 4.2644x vs baseline; 4.2644x over previous
import jax
import jax.numpy as jnp
from jax import lax
from jax.experimental import pallas as pl
from jax.experimental.pallas import tpu as pltpu

N_DEV = 8
SQ = 2048
SKV_SHARD = 2048
HQ = 8
DH = 128
DM = HQ * DH
CH = 256
NCH = SQ // CH
BB = 128
WIN = 512
SCALE = 0.08838834764831843
NEG = -1e9


def _expand_heads(s, rows):
    return jnp.broadcast_to(s[:, :, None], (rows, HQ, DH)).reshape(rows, DM)


def kernel(x, Wq, K_ext, V_ext, Wo):
    q = jnp.dot(x[0].astype(jnp.bfloat16), Wq.astype(jnp.bfloat16),
                preferred_element_type=jnp.float32)
    q_bf = (q * SCALE).astype(jnp.bfloat16)
    k_bf = K_ext[0].reshape(SKV_SHARD, DM).astype(jnp.bfloat16)
    v_bf = V_ext[0].reshape(SKV_SHARD, DM).astype(jnp.bfloat16)
    wo_bf = Wo.astype(jnp.bfloat16)

    def body(q_ref, k_ref, v_ref, wo_ref, out_ref,
             band_ctx, g_sacc, g_sl, g_racc, g_rl, kvs_k, kvs_v,
             band_snd, band_snd_l, band_rcv,
             g_sa, g_ra, g_sla, g_rla, kv_snd, kv_rcv):
        my = lax.axis_index("i")

        barrier = pltpu.get_barrier_semaphore()
        for j in range(1, N_DEV):
            pl.semaphore_signal(barrier, inc=1,
                                device_id=(lax.rem(my + j, N_DEV),),
                                device_id_type=pl.DeviceIdType.MESH)
        pl.semaphore_wait(barrier, N_DEV - 1)

        ga_cols, gl_cols = [], []
        for h in range(HQ):
            hs = slice(h * DH, (h + 1) * DH)
            s = lax.dot_general(q_ref[0:32, hs], k_ref[:, hs],
                                (((1,), (1,)), ((), ())),
                                preferred_element_type=jnp.float32)
            w = jnp.exp(s)
            gl_cols.append(jnp.sum(w, axis=1, keepdims=True))
            ga_cols.append(lax.dot_general(w.astype(jnp.bfloat16),
                                           v_ref[:, hs],
                                           (((1,), (0,)), ((), ())),
                                           preferred_element_type=jnp.float32))
        gacc = jnp.concatenate(ga_cols, axis=1)
        gl = jnp.concatenate(gl_cols, axis=1)
        g_sacc[...] = gacc.astype(jnp.bfloat16)
        g_sl[:, 0:HQ] = gl
        g_sl[:, HQ:2 * HQ] = gl

        g_descs = []
        for j in range(1, N_DEV):
            tgt = (lax.rem(my + j, N_DEV),)
            da = pltpu.make_async_remote_copy(
                src_ref=g_sacc, dst_ref=g_racc.at[j - 1],
                send_sem=g_sa.at[j - 1], recv_sem=g_ra.at[j - 1],
                device_id=tgt, device_id_type=pl.DeviceIdType.MESH)
            dl = pltpu.make_async_remote_copy(
                src_ref=g_sl, dst_ref=g_rl.at[j - 1],
                send_sem=g_sla.at[j - 1], recv_sem=g_rla.at[j - 1],
                device_id=tgt, device_id_type=pl.DeviceIdType.MESH)
            da.start()
            dl.start()
            g_descs.append((da, dl))

        @pl.when(my == 1)
        def _():
            dk = pltpu.make_async_remote_copy(
                src_ref=k_ref.at[pl.ds(0, BB)], dst_ref=kvs_k,
                send_sem=kv_snd.at[0], recv_sem=kv_rcv.at[0],
                device_id=(0,), device_id_type=pl.DeviceIdType.MESH)
            dv = pltpu.make_async_remote_copy(
                src_ref=v_ref.at[pl.ds(0, BB)], dst_ref=kvs_v,
                send_sem=kv_snd.at[1], recv_sem=kv_rcv.at[1],
                device_id=(0,), device_id_type=pl.DeviceIdType.MESH)
            dk.start()
            dv.start()
            dk.wait_send()
            dv.wait_send()

        def band_pieces(b_rows0, ws, extra):
            qiw = lax.broadcasted_iota(jnp.int32, (BB, WIN), 0) + b_rows0
            kiw = lax.broadcasted_iota(jnp.int32, (BB, WIN), 1) + ws
            mask_a = (qiw >= 32) & ((jnp.abs(qiw - kiw) <= 128) | (kiw < 32))
            qib = lax.broadcasted_iota(jnp.int32, (BB, BB), 0) + b_rows0
            kib = lax.broadcasted_iota(jnp.int32, (BB, BB), 1)
            mask_b = (qib >= 32) & (kib < 32) & (kib < ws)
            mask_c = jnp.abs(qib - (SKV_SHARD + kib)) <= 128
            if isinstance(b_rows0, int):
                rows = slice(b_rows0, b_rows0 + BB)
                kwin = slice(ws, ws + WIN)
            else:
                rows = pl.ds(b_rows0, BB)
                kwin = pl.ds(ws, WIN)
            for h in range(HQ):
                hs = slice(h * DH, (h + 1) * DH)
                sa = lax.dot_general(q_ref[rows, hs], k_ref[kwin, hs],
                                     (((1,), (1,)), ((), ())),
                                     preferred_element_type=jnp.float32)
                sb = lax.dot_general(q_ref[rows, hs], k_ref[0:BB, hs],
                                     (((1,), (1,)), ((), ())),
                                     preferred_element_type=jnp.float32)
                wa = jnp.exp(jnp.where(mask_a, sa, NEG))
                wb = jnp.exp(jnp.where(mask_b, sb, NEG))
                l = (jnp.sum(wa, axis=1, keepdims=True)
                     + jnp.sum(wb, axis=1, keepdims=True))
                a = (lax.dot_general(wa.astype(jnp.bfloat16), v_ref[kwin, hs],
                                     (((1,), (0,)), ((), ())),
                                     preferred_element_type=jnp.float32)
                     + lax.dot_general(wb.astype(jnp.bfloat16), v_ref[0:BB, hs],
                                       (((1,), (0,)), ((), ())),
                                       preferred_element_type=jnp.float32))
                if extra:
                    sc = lax.dot_general(q_ref[rows, hs], kvs_k[:, hs],
                                         (((1,), (1,)), ((), ())),
                                         preferred_element_type=jnp.float32)
                    wc = jnp.exp(jnp.where(mask_c, sc, NEG))
                    l = l + jnp.sum(wc, axis=1, keepdims=True)
                    a = a + lax.dot_general(wc.astype(jnp.bfloat16),
                                            kvs_v[:, hs],
                                            (((1,), (0,)), ((), ())),
                                            preferred_element_type=jnp.float32)
                ctx = a / jnp.where(l > 0, l, 1.0)
                band_ctx[rows, hs] = ctx.astype(jnp.bfloat16)

        def band_block(b, _):
            wblk = jnp.minimum(jnp.maximum(b - 1, 0), (SKV_SHARD - WIN) // BB)
            band_pieces(b * BB, wblk * BB, extra=False)
            return 0

        lax.fori_loop(0, jnp.where(my == 0, 14, 0), band_block, 0)

        @pl.when(my == 0)
        def _():
            rk = pltpu.make_async_remote_copy(
                src_ref=kvs_k, dst_ref=kvs_k, send_sem=kv_snd.at[0],
                recv_sem=kv_rcv.at[0], device_id=(1,),
                device_id_type=pl.DeviceIdType.MESH)
            rv = pltpu.make_async_remote_copy(
                src_ref=kvs_v, dst_ref=kvs_v, send_sem=kv_snd.at[1],
                recv_sem=kv_rcv.at[1], device_id=(1,),
                device_id_type=pl.DeviceIdType.MESH)
            rk.wait_recv()
            rv.wait_recv()
            band_pieces(14 * BB, SKV_SHARD - WIN, extra=True)
            band_pieces(15 * BB, SKV_SHARD - WIN, extra=True)

        for da, dl in g_descs:
            da.wait_recv()
            dl.wait_recv()
        for s in range(N_DEV - 1):
            gacc = gacc + g_racc[s].astype(jnp.float32)
            gl = gl + g_rl[s, :, 0:HQ]
        gctx = (gacc * _expand_heads(1.0 / gl, 32)).astype(jnp.bfloat16)

        is_r_send = my <= 3
        is_l_send = (my == 0) | (my >= 6)
        fwd_r, fwd_l = [], []
        for c in range(NCH):
            rows = pl.ds(c * CH, CH)
            desc_r = pltpu.make_async_remote_copy(
                src_ref=band_ctx.at[rows], dst_ref=band_ctx.at[rows],
                send_sem=band_snd.at[c], recv_sem=band_rcv.at[c],
                device_id=(lax.rem(my + 1, N_DEV),),
                device_id_type=pl.DeviceIdType.MESH)
            desc_l = pltpu.make_async_remote_copy(
                src_ref=band_ctx.at[rows], dst_ref=band_ctx.at[rows],
                send_sem=band_snd_l.at[c], recv_sem=band_rcv.at[c],
                device_id=(lax.rem(my + N_DEV - 1, N_DEV),),
                device_id_type=pl.DeviceIdType.MESH)
            fwd_r.append(desc_r)
            fwd_l.append(desc_l)

            @pl.when(my != 0)
            def _(desc=desc_r):
                desc.wait_recv()

            @pl.when(is_r_send)
            def _(desc=desc_r):
                desc.start()

            @pl.when(is_l_send)
            def _(desc=desc_l):
                desc.start()

            ctx_chunk = band_ctx[rows, :]
            if c == 0:
                ctx_chunk = jnp.concatenate(
                    [gctx, ctx_chunk[32:, :]], axis=0)
            out_ref[rows, :] = lax.dot_general(
                ctx_chunk, wo_ref[...], (((1,), (0,)), ((), ())),
                preferred_element_type=jnp.float32)

        for da, dl in g_descs:
            da.wait_send()
            dl.wait_send()

        @pl.when(is_r_send)
        def _():
            for desc in fwd_r:
                desc.wait_send()

        @pl.when(is_l_send)
        def _():
            for desc in fwd_l:
                desc.wait_send()

    out = pl.pallas_call(
        body,
        out_shape=jax.ShapeDtypeStruct((SQ, DM), jnp.float32),
        in_specs=[pl.BlockSpec(memory_space=pltpu.VMEM)] * 4,
        out_specs=pl.BlockSpec(memory_space=pltpu.VMEM),
        scratch_shapes=[
            pltpu.VMEM((SQ, DM), jnp.bfloat16),
            pltpu.VMEM((32, DM), jnp.bfloat16),
            pltpu.VMEM((32, 2 * HQ), jnp.float32),
            pltpu.VMEM((N_DEV - 1, 32, DM), jnp.bfloat16),
            pltpu.VMEM((N_DEV - 1, 32, 2 * HQ), jnp.float32),
            pltpu.VMEM((BB, DM), jnp.bfloat16),
            pltpu.VMEM((BB, DM), jnp.bfloat16),
            pltpu.SemaphoreType.DMA((NCH,)),
            pltpu.SemaphoreType.DMA((NCH,)),
            pltpu.SemaphoreType.DMA((NCH,)),
            pltpu.SemaphoreType.DMA((N_DEV - 1,)),
            pltpu.SemaphoreType.DMA((N_DEV - 1,)),
            pltpu.SemaphoreType.DMA((N_DEV - 1,)),
            pltpu.SemaphoreType.DMA((N_DEV - 1,)),
            pltpu.SemaphoreType.DMA((2,)),
            pltpu.SemaphoreType.DMA((2,)),
        ],
        compiler_params=pltpu.CompilerParams(
            collective_id=0, vmem_limit_bytes=60 * 1024 * 1024),
    )(q_bf, k_bf, v_bf, wo_bf)
    return out.reshape(1, SQ, DM)
